# baseline (speedup 1.0000x reference)
import functools

import jax
import jax.numpy as jnp
from jax import lax
from jax.experimental import pallas as pl
from jax.experimental.pallas import tpu as pltpu
from jax.experimental.pallas import tpu_sc as plsc

_B = 16
_H = 12
_S = 577
_L = 16
_NCHUNK = 37
_SP = _NCHUNK * _L


def _tec_work(x_ref, out_ref, model, b, rows, mv16, outv, sem):
    cp = pltpu.make_async_copy(x_ref.at[model, b], rows, sem)
    cp.start()
    cp.wait()

    lane = lax.iota(jnp.int32, _L)

    def head_sum(ci):
        acc = rows[0, pl.ds(ci * _L, _L)]
        for h in range(1, _H):
            acc = acc + rows[h, pl.ds(ci * _L, _L)]
        return acc

    mv16[...] = head_sum(1)
    v19 = plsc.load_gather(mv16, [jnp.full((_L,), 3, jnp.int32)])
    v20 = plsc.load_gather(mv16, [jnp.full((_L,), 4, jnp.int32)])
    v21 = plsc.load_gather(mv16, [jnp.full((_L,), 5, jnp.int32)])

    c19 = c20 = c21 = None

    def acc(cacc, hit):
        n = plsc.all_reduce_population_count(hit)
        return n if cacc is None else cacc + n

    for ci in range(_NCHUNK):
        x = head_sum(ci)
        gidx = ci * _L + lane
        if ci <= 1:
            c19 = acc(c19, (x < v19) | ((x == v19) & (gidx < 19)))
            c20 = acc(c20, (x < v20) | ((x == v20) & (gidx < 20)))
            c21 = acc(c21, (x < v21) | ((x == v21) & (gidx < 21)))
        elif ci == _NCHUNK - 1:
            valid = gidx < _S
            c19 = acc(c19, (x < v19) & valid)
            c20 = acc(c20, (x < v20) & valid)
            c21 = acc(c21, (x < v21) & valid)
        else:
            c19 = acc(c19, x < v19)
            c20 = acc(c20, x < v20)
            c21 = acc(c21, x < v21)

    rank = jnp.where(lane == 0, c19, jnp.where(lane == 1, c20, c21))
    rank = jnp.where(lane < 3, rank, 0)
    outv[...] = rank.astype(jnp.float32)
    pltpu.make_async_copy(outv, out_ref.at[model, b, :], sem).start()
    pltpu.make_async_copy(outv, out_ref.at[model, b, :], sem).wait()


def _sc_body(x_ref, out_ref, rows, mv16, outv, sem):
    core = lax.axis_index("c")
    b = lax.axis_index("s")

    @pl.when(core == 0)
    def _():
        _tec_work(x_ref, out_ref, 0, b, rows, mv16, outv, sem)

    @pl.when(core == 1)
    def _():
        _tec_work(x_ref, out_ref, 1, b, rows, mv16, outv, sem)


_sc_call = functools.partial(
    pl.kernel,
    out_type=jax.ShapeDtypeStruct((2, _B, _L), jnp.float32),
    mesh=plsc.VectorSubcoreMesh(core_axis_name="c", subcore_axis_name="s"),
    compiler_params=pltpu.CompilerParams(
        use_tc_tiling_on_sc=False, needs_layout_passes=False,
        skip_device_barrier=True),
    scratch_types=[
        pltpu.VMEM((_H, _SP), jnp.float32),
        pltpu.VMEM((_L,), jnp.float32),
        pltpu.VMEM((_L,), jnp.float32),
        pltpu.SemaphoreType.DMA,
    ],
)(_sc_body)


def kernel(attn1, attn2):
    x = jnp.stack((attn1[:, :, 0, :], attn2[:, :, 0, :]))
    x = jnp.pad(x, ((0, 0), (0, 0), (0, 0), (0, _SP - _S)))
    ranks = _sc_call(x)
    return (ranks[0, :, :3] + ranks[1, :, :3]) * 0.5 + 1.0


# speedup vs baseline: 1.0503x; 1.0503x over previous
import functools

import jax
import jax.numpy as jnp
from jax import lax
from jax.experimental import pallas as pl
from jax.experimental.pallas import tpu as pltpu
from jax.experimental.pallas import tpu_sc as plsc

_B = 16
_H = 12
_S = 577
_L = 16
_NCHUNK = 37
_SP = _NCHUNK * _L


def _sc_body(x_ref, out_ref, rows, mv16, outv, sem):
    model = lax.axis_index("c")
    b = lax.axis_index("s")

    cp = pltpu.make_async_copy(x_ref.at[model, b], rows, sem)
    cp.start()
    cp.wait()

    lane = lax.iota(jnp.int32, _L)

    def head_sum(ci):
        acc = rows[0, pl.ds(ci * _L, _L)]
        for h in range(1, _H):
            acc = acc + rows[h, pl.ds(ci * _L, _L)]
        return acc

    mv16[...] = head_sum(1)
    v19 = plsc.load_gather(mv16, [jnp.full((_L,), 3, jnp.int32)])
    v20 = plsc.load_gather(mv16, [jnp.full((_L,), 4, jnp.int32)])
    v21 = plsc.load_gather(mv16, [jnp.full((_L,), 5, jnp.int32)])

    def count_chunk(ci, carry):
        c19, c20, c21 = carry
        x = head_sum(ci)
        gidx = ci * _L + lane
        valid = gidx < _S

        def cnt(vk, k, cacc):
            hit = ((x < vk) & valid) | ((x == vk) & (gidx < k))
            return cacc + plsc.all_reduce_population_count(hit)

        return (cnt(v19, 19, c19), cnt(v20, 20, c20), cnt(v21, 21, c21))

    zi = jnp.zeros((_L,), jnp.int32)
    c19, c20, c21 = lax.fori_loop(0, _NCHUNK, count_chunk, (zi, zi, zi))

    rank = jnp.where(lane == 0, c19, jnp.where(lane == 1, c20, c21))
    rank = jnp.where(lane < 3, rank, 0)
    outv[...] = rank.astype(jnp.float32)
    pltpu.make_async_copy(outv, out_ref.at[model, b, :], sem).start()
    pltpu.make_async_copy(outv, out_ref.at[model, b, :], sem).wait()


_sc_call = functools.partial(
    pl.kernel,
    out_type=jax.ShapeDtypeStruct((2, _B, _L), jnp.float32),
    mesh=plsc.VectorSubcoreMesh(core_axis_name="c", subcore_axis_name="s"),
    compiler_params=pltpu.CompilerParams(
        use_tc_tiling_on_sc=False, needs_layout_passes=False,
        skip_device_barrier=True),
    scratch_types=[
        pltpu.VMEM((_H, _SP), jnp.float32),
        pltpu.VMEM((_L,), jnp.float32),
        pltpu.VMEM((_L,), jnp.float32),
        pltpu.SemaphoreType.DMA,
    ],
)(_sc_body)


def kernel(attn1, attn2):
    x = jnp.stack((attn1[:, :, 0, :], attn2[:, :, 0, :]))
    x = jnp.pad(x, ((0, 0), (0, 0), (0, 0), (0, _SP - _S)))
    ranks = _sc_call(x)
    return (ranks[0, :, :3] + ranks[1, :, :3]) * 0.5 + 1.0
